# baseline (speedup 1.0000x reference)
import functools

import jax
import jax.numpy as jnp
from jax import lax
from jax.experimental import pallas as pl
from jax.experimental.pallas import tpu as pltpu
from jax.experimental.pallas import tpu_sc as plsc

IN_DIM = 128
HIDDEN_DIM = 128
OUT_DIM = 128
N_NODES = 10000
N_EDGES = 320000
N_CH = 4
N_PAIR = 2

SC_CORES = 2
SC_SUBCORES = 16
NW = SC_CORES * SC_SUBCORES
EPW = N_EDGES // NW
CHUNK = 80
N_CHUNKS = EPW // CHUNK
_MAIN_PAIRS = (N_CHUNKS - 1) // 2


_PRE_BN = 2000


def _pre_body(emb_ref, w1s_ref, w1d_ref, b1_ref, a_ref, b_ref):
    emb = emb_ref[0]
    a_ref[0] = (
        jnp.dot(emb, w1s_ref[...], preferred_element_type=jnp.float32)
        + b1_ref[...]
    ).astype(jnp.bfloat16)
    b_ref[0] = jnp.dot(
        emb, w1d_ref[...], preferred_element_type=jnp.float32
    ).astype(jnp.bfloat16)


def _precompute(mc_embeddings, w1s_t, w1d_t, b1_row):
    grid = (N_CH, N_NODES // _PRE_BN)
    return pl.pallas_call(
        _pre_body,
        grid=grid,
        in_specs=[
            pl.BlockSpec((1, _PRE_BN, IN_DIM), lambda c, n: (c, n, 0)),
            pl.BlockSpec((IN_DIM, HIDDEN_DIM), lambda c, n: (0, 0)),
            pl.BlockSpec((IN_DIM, HIDDEN_DIM), lambda c, n: (0, 0)),
            pl.BlockSpec((1, HIDDEN_DIM), lambda c, n: (0, 0)),
        ],
        out_specs=[
            pl.BlockSpec((1, _PRE_BN, HIDDEN_DIM), lambda c, n: (c, n, 0)),
            pl.BlockSpec((1, _PRE_BN, HIDDEN_DIM), lambda c, n: (c, n, 0)),
        ],
        out_shape=[
            jax.ShapeDtypeStruct((N_CH, N_NODES, HIDDEN_DIM), jnp.bfloat16),
            jax.ShapeDtypeStruct((N_CH, N_NODES, HIDDEN_DIM), jnp.bfloat16),
        ],
    )(mc_embeddings, w1s_t, w1d_t, b1_row)


def _sc_body(a0_hbm, a1_hbm, b0_hbm, b1_hbm, src_hbm, dst_hbm, p0, p1,
             idx_s, idx_d, buf_a0, buf_a1, buf_b0, buf_b1, res0, res1,
             sem_g0, sem_g1, sem_s0, sem_s1):
    cid = lax.axis_index("c")
    sid = lax.axis_index("s")
    wid = sid * SC_CORES + cid
    base = wid * EPW
    a_tabs = (a0_hbm, a1_hbm)
    b_tabs = (b0_hbm, b1_hbm)
    outs = (p0, p1)
    buf_a = (buf_a0, buf_a1)
    buf_b = (buf_b0, buf_b1)
    res = (res0, res1)
    sem_g = (sem_g0, sem_g1)
    sem_s = (sem_s0, sem_s1)

    pltpu.sync_copy(src_hbm.at[pl.ds(base, EPW)], idx_s)
    pltpu.sync_copy(dst_hbm.at[pl.ds(base, EPW)], idx_d)

    def issue_gathers(j, cp):
        isl = idx_s.at[pl.ds(j * CHUNK, CHUNK)]
        idl = idx_d.at[pl.ds(j * CHUNK, CHUNK)]
        pltpu.async_copy(a_tabs[cp].at[isl], buf_a[cp], sem_g[cp])
        pltpu.async_copy(b_tabs[cp].at[idl], buf_b[cp], sem_g[cp])

    def wait_gathers(cp):
        isl = idx_s.at[pl.ds(0, CHUNK)]
        idl = idx_d.at[pl.ds(0, CHUNK)]
        pltpu.make_async_copy(a_tabs[0].at[isl], buf_a[cp], sem_g[cp]).wait()
        pltpu.make_async_copy(b_tabs[0].at[idl], buf_b[cp], sem_g[cp]).wait()

    def wait_scatter(cp):
        pltpu.make_async_copy(
            res[cp], outs[0].at[pl.ds(base, CHUNK)], sem_s[cp]
        ).wait()

    def step(j, cp, jn, cpn, guard_j2=None, last=False):
        q = 1 - cp
        if not last:
            if guard_j2 is None:
                wait_scatter(q)
            else:
                @pl.when(guard_j2 > 0)
                def _():
                    wait_scatter(q)
            issue_gathers(jn, cpn)
        wait_gathers(cp)
        a = buf_a[cp]
        b = buf_b[cp]
        r_buf = res[cp]

        def add_body(r, carry):
            for c in range(HIDDEN_DIM // 16):
                va = plsc.bitcast(a[r, pl.ds(c * 16, 16)], jnp.bfloat16)
                vb = plsc.bitcast(b[r, pl.ds(c * 16, 16)], jnp.bfloat16)
                r_buf[r, pl.ds(c * 16, 16)] = plsc.bitcast(
                    va + vb, jnp.int32
                )
            return carry

        lax.fori_loop(0, CHUNK, add_body, 0)
        pltpu.async_copy(
            r_buf, outs[cp].at[pl.ds(base + j * CHUNK, CHUNK)], sem_s[cp]
        )

    issue_gathers(0, 0)

    def body2(j2, carry):
        for jp in range(2):
            j = 2 * j2 + jp
            for cp in range(N_PAIR):
                cpn = (cp + 1) % N_PAIR
                jn = j + (1 if cp == N_PAIR - 1 else 0)
                guard = j2 if (jp == 0 and cp == 0) else None
                step(j, cp, jn, cpn, guard_j2=guard)
        return carry

    lax.fori_loop(0, _MAIN_PAIRS, body2, 0)

    j_tail = N_CHUNKS - 1
    for cp in range(N_PAIR):
        cpn = (cp + 1) % N_PAIR
        step(j_tail, cp, j_tail, cpn, last=(cp == N_PAIR - 1))

    wait_scatter(0)
    wait_scatter(1)


def _sc_gather(a01, a23, b01, b23, src, dst):
    mesh = plsc.VectorSubcoreMesh(
        core_axis_name="c", subcore_axis_name="s",
        num_cores=SC_CORES, num_subcores=SC_SUBCORES,
    )
    out_t = [
        jax.ShapeDtypeStruct((N_EDGES, HIDDEN_DIM), jnp.int32)
    ] * N_PAIR
    f = pl.kernel(
        _sc_body,
        out_type=out_t,
        mesh=mesh,
        compiler_params=pltpu.CompilerParams(needs_layout_passes=False),
        scratch_types=[
            pltpu.VMEM((EPW,), jnp.int32),
            pltpu.VMEM((EPW,), jnp.int32),
            pltpu.VMEM((CHUNK, HIDDEN_DIM), jnp.int32),
            pltpu.VMEM((CHUNK, HIDDEN_DIM), jnp.int32),
            pltpu.VMEM((CHUNK, HIDDEN_DIM), jnp.int32),
            pltpu.VMEM((CHUNK, HIDDEN_DIM), jnp.int32),
            pltpu.VMEM((CHUNK, HIDDEN_DIM), jnp.int32),
            pltpu.VMEM((CHUNK, HIDDEN_DIM), jnp.int32),
            pltpu.SemaphoreType.DMA,
            pltpu.SemaphoreType.DMA,
            pltpu.SemaphoreType.DMA,
            pltpu.SemaphoreType.DMA,
        ],
    )
    return f(a01, a23, b01, b23, src, dst)


_MM_BE = 2000


def _mm_body(p01, p23, w2e_ref, w2o_ref, b2_ref, o0, o1, o2, o3):
    w2e = w2e_ref[...]
    w2o = w2o_ref[...]
    b2v = b2_ref[...]
    zero = jnp.zeros((), jnp.bfloat16)
    for p_ref, o_even, o_odd in ((p01, o0, o1), (p23, o2, o3)):
        h = jnp.maximum(p_ref[...], zero)
        o_even[...] = (
            jnp.dot(h, w2e, preferred_element_type=jnp.float32) + b2v
        )
        o_odd[...] = (
            jnp.dot(h, w2o, preferred_element_type=jnp.float32) + b2v
        )


def _final_mm(p01, p23, w2_even, w2_odd, b2_row):
    grid = (N_EDGES // _MM_BE,)
    pair_spec = pl.BlockSpec((_MM_BE, 2 * HIDDEN_DIM), lambda e: (e, 0))
    return pl.pallas_call(
        _mm_body,
        grid=grid,
        in_specs=[
            pair_spec,
            pair_spec,
            pl.BlockSpec((2 * HIDDEN_DIM, OUT_DIM), lambda e: (0, 0)),
            pl.BlockSpec((2 * HIDDEN_DIM, OUT_DIM), lambda e: (0, 0)),
            pl.BlockSpec((1, OUT_DIM), lambda e: (0, 0)),
        ],
        out_specs=[pl.BlockSpec((_MM_BE, OUT_DIM), lambda e: (e, 0))] * N_CH,
        out_shape=[jax.ShapeDtypeStruct((N_EDGES, OUT_DIM), jnp.float32)] * N_CH,
    )(p01, p23, w2_even, w2_odd, b2_row)


def _pack_pair(tab_bf16, lo, hi):
    pair = jnp.stack([tab_bf16[lo], tab_bf16[hi]], axis=-1)
    return lax.bitcast_convert_type(pair, jnp.int32)


def kernel(edge_index, mc_embeddings, W1, b1, W2, b2):
    w1s_t = W1[:, :IN_DIM].T
    w1d_t = W1[:, IN_DIM:].T
    a_bf, b_bf = _precompute(
        mc_embeddings, w1s_t, w1d_t, b1.reshape(1, HIDDEN_DIM)
    )
    a01 = _pack_pair(a_bf, 0, 1)
    a23 = _pack_pair(a_bf, 2, 3)
    b01 = _pack_pair(b_bf, 0, 1)
    b23 = _pack_pair(b_bf, 2, 3)
    p01, p23 = _sc_gather(a01, a23, b01, b23, edge_index[0], edge_index[1])
    p01 = lax.bitcast_convert_type(p01, jnp.bfloat16).reshape(
        N_EDGES, 2 * HIDDEN_DIM
    )
    p23 = lax.bitcast_convert_type(p23, jnp.bfloat16).reshape(
        N_EDGES, 2 * HIDDEN_DIM
    )

    w2t = W2.T.astype(jnp.bfloat16)
    zeros = jnp.zeros_like(w2t)
    w2_even = jnp.stack([w2t, zeros], axis=1).reshape(2 * HIDDEN_DIM, OUT_DIM)
    w2_odd = jnp.stack([zeros, w2t], axis=1).reshape(2 * HIDDEN_DIM, OUT_DIM)

    outs = _final_mm(p01, p23, w2_even, w2_odd, b2.reshape(1, OUT_DIM))
    return tuple(outs)


# speedup vs baseline: 4.2789x; 4.2789x over previous
import functools

import jax
import jax.numpy as jnp
from jax import lax
from jax.experimental import pallas as pl
from jax.experimental.pallas import tpu as pltpu
from jax.experimental.pallas import tpu_sc as plsc

IN_DIM = 128
HIDDEN_DIM = 128
OUT_DIM = 128
N_NODES = 10000
N_EDGES = 320000
N_CH = 4
N_PAIR = 2

SC_CORES = 2
SC_SUBCORES = 16
NW = SC_CORES * SC_SUBCORES
EPW = N_EDGES // NW
CHUNK = 80
N_CHUNKS = EPW // CHUNK
_MAIN_PAIRS = (N_CHUNKS - 1) // 2


def _rtne_bf16_bits(x_f32):
    b = lax.bitcast_convert_type(x_f32, jnp.int32)
    return (b + jnp.int32(0x7FFF) + ((b >> 16) & jnp.int32(1))) >> 16


_PRE_BN = 2000


def _pre_body(emb_e_ref, emb_o_ref, w1s_ref, w1d_ref, b1_ref, a_ref, b_ref):
    emb_e = emb_e_ref[0]
    emb_o = emb_o_ref[0]
    w1s = w1s_ref[...]
    w1d = w1d_ref[...]
    b1v = b1_ref[...]
    mask = jnp.int32(0xFFFF)

    def pack(lo_f32, hi_f32):
        lo = _rtne_bf16_bits(lo_f32) & mask
        hi = _rtne_bf16_bits(hi_f32) << 16
        return lo | hi

    a_e = jnp.dot(emb_e, w1s, preferred_element_type=jnp.float32) + b1v
    a_o = jnp.dot(emb_o, w1s, preferred_element_type=jnp.float32) + b1v
    b_e = jnp.dot(emb_e, w1d, preferred_element_type=jnp.float32)
    b_o = jnp.dot(emb_o, w1d, preferred_element_type=jnp.float32)
    a_ref[0] = pack(a_e, a_o)
    b_ref[0] = pack(b_e, b_o)


def _precompute(mc_embeddings, w1s_t, w1d_t, b1_row):
    grid = (N_PAIR, N_NODES // _PRE_BN)
    emb_spec_e = pl.BlockSpec((1, _PRE_BN, IN_DIM), lambda p, n: (2 * p, n, 0))
    emb_spec_o = pl.BlockSpec(
        (1, _PRE_BN, IN_DIM), lambda p, n: (2 * p + 1, n, 0)
    )
    return pl.pallas_call(
        _pre_body,
        grid=grid,
        in_specs=[
            emb_spec_e,
            emb_spec_o,
            pl.BlockSpec((IN_DIM, HIDDEN_DIM), lambda p, n: (0, 0)),
            pl.BlockSpec((IN_DIM, HIDDEN_DIM), lambda p, n: (0, 0)),
            pl.BlockSpec((1, HIDDEN_DIM), lambda p, n: (0, 0)),
        ],
        out_specs=[
            pl.BlockSpec((1, _PRE_BN, HIDDEN_DIM), lambda p, n: (p, n, 0)),
            pl.BlockSpec((1, _PRE_BN, HIDDEN_DIM), lambda p, n: (p, n, 0)),
        ],
        out_shape=[
            jax.ShapeDtypeStruct((N_PAIR, N_NODES, HIDDEN_DIM), jnp.int32),
            jax.ShapeDtypeStruct((N_PAIR, N_NODES, HIDDEN_DIM), jnp.int32),
        ],
    )(mc_embeddings, mc_embeddings, w1s_t, w1d_t, b1_row)


def _sc_body(a_hbm, b_hbm, src_hbm, dst_hbm, p0, p1,
             idx_s, idx_d, buf_a0, buf_a1, buf_b0, buf_b1, res0, res1,
             sem_g0, sem_g1, sem_s0, sem_s1):
    cid = lax.axis_index("c")
    sid = lax.axis_index("s")
    wid = sid * SC_CORES + cid
    base = wid * EPW
    outs = (p0, p1)
    buf_a = (buf_a0, buf_a1)
    buf_b = (buf_b0, buf_b1)
    res = (res0, res1)
    sem_g = (sem_g0, sem_g1)
    sem_s = (sem_s0, sem_s1)

    pltpu.sync_copy(src_hbm.at[pl.ds(base, EPW)], idx_s)
    pltpu.sync_copy(dst_hbm.at[pl.ds(base, EPW)], idx_d)

    def issue_gathers(j, cp):
        isl = idx_s.at[pl.ds(j * CHUNK, CHUNK)]
        idl = idx_d.at[pl.ds(j * CHUNK, CHUNK)]
        pltpu.async_copy(a_hbm.at[cp].at[isl], buf_a[cp], sem_g[cp])
        pltpu.async_copy(b_hbm.at[cp].at[idl], buf_b[cp], sem_g[cp])

    def wait_gathers(cp):
        isl = idx_s.at[pl.ds(0, CHUNK)]
        idl = idx_d.at[pl.ds(0, CHUNK)]
        pltpu.make_async_copy(a_hbm.at[0].at[isl], buf_a[cp], sem_g[cp]).wait()
        pltpu.make_async_copy(b_hbm.at[0].at[idl], buf_b[cp], sem_g[cp]).wait()

    def wait_scatter(cp):
        pltpu.make_async_copy(
            res[cp], outs[0].at[pl.ds(base, CHUNK)], sem_s[cp]
        ).wait()

    def step(j, cp, jn, cpn, guard_j2=None, last=False):
        q = 1 - cp
        if not last:
            if guard_j2 is None:
                wait_scatter(q)
            else:
                @pl.when(guard_j2 > 0)
                def _():
                    wait_scatter(q)
            issue_gathers(jn, cpn)
        wait_gathers(cp)
        a = buf_a[cp]
        b = buf_b[cp]
        r_buf = res[cp]

        def add_body(r, carry):
            for c in range(HIDDEN_DIM // 16):
                va = plsc.bitcast(a[r, pl.ds(c * 16, 16)], jnp.bfloat16)
                vb = plsc.bitcast(b[r, pl.ds(c * 16, 16)], jnp.bfloat16)
                r_buf[r, pl.ds(c * 16, 16)] = plsc.bitcast(
                    va + vb, jnp.int32
                )
            return carry

        lax.fori_loop(0, CHUNK, add_body, 0)
        pltpu.async_copy(
            r_buf, outs[cp].at[pl.ds(base + j * CHUNK, CHUNK)], sem_s[cp]
        )

    issue_gathers(0, 0)

    def body2(j2, carry):
        for jp in range(2):
            j = 2 * j2 + jp
            for cp in range(N_PAIR):
                cpn = (cp + 1) % N_PAIR
                jn = j + (1 if cp == N_PAIR - 1 else 0)
                guard = j2 if (jp == 0 and cp == 0) else None
                step(j, cp, jn, cpn, guard_j2=guard)
        return carry

    lax.fori_loop(0, _MAIN_PAIRS, body2, 0)

    j_tail = N_CHUNKS - 1
    for cp in range(N_PAIR):
        cpn = (cp + 1) % N_PAIR
        step(j_tail, cp, j_tail, cpn, last=(cp == N_PAIR - 1))

    wait_scatter(0)
    wait_scatter(1)


def _sc_gather(a_pack, b_pack, src, dst):
    mesh = plsc.VectorSubcoreMesh(
        core_axis_name="c", subcore_axis_name="s",
        num_cores=SC_CORES, num_subcores=SC_SUBCORES,
    )
    out_t = [
        jax.ShapeDtypeStruct((N_EDGES, HIDDEN_DIM), jnp.int32)
    ] * N_PAIR
    f = pl.kernel(
        _sc_body,
        out_type=out_t,
        mesh=mesh,
        compiler_params=pltpu.CompilerParams(needs_layout_passes=False),
        scratch_types=[
            pltpu.VMEM((EPW,), jnp.int32),
            pltpu.VMEM((EPW,), jnp.int32),
            pltpu.VMEM((CHUNK, HIDDEN_DIM), jnp.int32),
            pltpu.VMEM((CHUNK, HIDDEN_DIM), jnp.int32),
            pltpu.VMEM((CHUNK, HIDDEN_DIM), jnp.int32),
            pltpu.VMEM((CHUNK, HIDDEN_DIM), jnp.int32),
            pltpu.VMEM((CHUNK, HIDDEN_DIM), jnp.int32),
            pltpu.VMEM((CHUNK, HIDDEN_DIM), jnp.int32),
            pltpu.SemaphoreType.DMA,
            pltpu.SemaphoreType.DMA,
            pltpu.SemaphoreType.DMA,
            pltpu.SemaphoreType.DMA,
        ],
    )
    return f(a_pack, b_pack, src, dst)


_MM_BE = 2000


def _mm_body(p01, p23, w2t_ref, b2_ref, o0, o1, o2, o3):
    w2t = w2t_ref[...]
    b2v = b2_ref[...]
    himask = jnp.int32(-65536)

    for p_ref, o_even, o_odd in ((p01, o0, o1), (p23, o2, o3)):
        x = p_ref[...]
        lo = lax.bitcast_convert_type(x << 16, jnp.float32)
        hi = lax.bitcast_convert_type(x & himask, jnp.float32)
        h_e = jnp.maximum(lo, 0.0).astype(jnp.bfloat16)
        h_o = jnp.maximum(hi, 0.0).astype(jnp.bfloat16)
        o_even[...] = (
            jnp.dot(h_e, w2t, preferred_element_type=jnp.float32) + b2v
        )
        o_odd[...] = (
            jnp.dot(h_o, w2t, preferred_element_type=jnp.float32) + b2v
        )


def _final_mm(p01, p23, w2_t, b2_row):
    grid = (N_EDGES // _MM_BE,)
    pair_spec = pl.BlockSpec((_MM_BE, HIDDEN_DIM), lambda e: (e, 0))
    return pl.pallas_call(
        _mm_body,
        grid=grid,
        in_specs=[
            pair_spec,
            pair_spec,
            pl.BlockSpec((HIDDEN_DIM, OUT_DIM), lambda e: (0, 0)),
            pl.BlockSpec((1, OUT_DIM), lambda e: (0, 0)),
        ],
        out_specs=[pl.BlockSpec((_MM_BE, OUT_DIM), lambda e: (e, 0))] * N_CH,
        out_shape=[jax.ShapeDtypeStruct((N_EDGES, OUT_DIM), jnp.float32)] * N_CH,
    )(p01, p23, w2_t, b2_row)


def kernel(edge_index, mc_embeddings, W1, b1, W2, b2):
    w1s_t = W1[:, :IN_DIM].T
    w1d_t = W1[:, IN_DIM:].T
    w2_t = W2.T.astype(jnp.bfloat16)
    a_pack, b_pack = _precompute(
        mc_embeddings, w1s_t, w1d_t, b1.reshape(1, HIDDEN_DIM)
    )
    p01, p23 = _sc_gather(a_pack, b_pack, edge_index[0], edge_index[1])
    outs = _final_mm(p01, p23, w2_t, b2.reshape(1, OUT_DIM))
    return tuple(outs)
